# baseline (speedup 1.0000x reference)
import functools

import jax
import jax.numpy as jnp
from jax import lax
from jax.experimental import pallas as pl
from jax.experimental.pallas import tpu as pltpu
from jax.experimental.pallas import tpu_sc as plsc

_NC = 2
_NS = 16
_NW = _NC * _NS
_L = 16
_K = 128
_D = 128
_DW = _D // 2
_GP = _DW // _L
_CPP = 160
_F = 128
_S = _CPP - _F
_FAST_CORE = 0


def _sc_link_predict(zp, zc, src2d, tgt2d, w_flat, b_vec):
    n_chunks = src2d.shape[0]

    mesh = plsc.VectorSubcoreMesh(core_axis_name="c", subcore_axis_name="s")

    @functools.partial(
        pl.kernel,
        mesh=mesh,
        out_type=jax.ShapeDtypeStruct((n_chunks, _K), jnp.float32),
        compiler_params=pltpu.CompilerParams(needs_layout_passes=False,
                                             use_tc_tiling_on_sc=False),
        scratch_types=[
            pltpu.VMEM((_F, _K), jnp.int32),
            pltpu.VMEM((_F, _K), jnp.int32),
            pltpu.VMEM((_K, _DW), jnp.int32),
            pltpu.VMEM((_K, _DW), jnp.int32),
            pltpu.VMEM((_K, _DW), jnp.int32),
            pltpu.VMEM((_K, _DW), jnp.int32),
            pltpu.VMEM((_F, _K), jnp.float32),
            pltpu.VMEM((_D,), jnp.float32),
            pltpu.VMEM((_L,), jnp.float32),
            pltpu.SemaphoreType.DMA,
            pltpu.SemaphoreType.DMA,
        ],
    )
    def body(zp_r, zc_r, src_r, tgt_r, w_r, b_r, out_r,
             idx_s_sl, idx_t_sl, rs0, rs1, rt0, rt1, out_sl, w_v, b_v,
             sem0, sem1):
        cid = lax.axis_index("c")
        sid = lax.axis_index("s")
        on_fast = cid == _FAST_CORE
        my_n = jnp.where(on_fast, _F, _S)
        start = pl.multiple_of(sid * _CPP + jnp.where(on_fast, 0, _F), 8)

        pltpu.sync_copy(w_r, w_v)
        pltpu.sync_copy(b_r, b_v)
        bvec = b_v[...]
        lane = lax.iota(jnp.int32, _L)
        wev = [w_v[pl.ds(g * _L, _L)] for g in range(_GP)]
        wov = [w_v[pl.ds(_DW + g * _L, _L)] for g in range(_GP)]

        @pl.when(on_fast)
        def _():
            pltpu.sync_copy(src_r.at[pl.ds(start, _F)], idx_s_sl)
            pltpu.sync_copy(tgt_r.at[pl.ds(start, _F)], idx_t_sl)

        @pl.when(jnp.logical_not(on_fast))
        def _():
            pltpu.sync_copy(src_r.at[pl.ds(start, _S)],
                            idx_s_sl.at[pl.ds(0, _S)])
            pltpu.sync_copy(tgt_r.at[pl.ds(start, _S)],
                            idx_t_sl.at[pl.ds(0, _S)])

        bufs = ((rs0, rt0, sem0), (rs1, rt1, sem1))

        def issue(c, b):
            rs, rt, sem = bufs[b]
            pltpu.async_copy(zp_r.at[idx_s_sl.at[c]], rs, sem)
            pltpu.async_copy(zc_r.at[idx_t_sl.at[c]], rt, sem)

        def wait(c, b):
            rs, rt, sem = bufs[b]
            pltpu.make_async_copy(zp_r.at[idx_s_sl.at[c]], rs, sem).wait()
            pltpu.make_async_copy(zc_r.at[idx_t_sl.at[c]], rt, sem).wait()

        def compute(c, b):
            rs, rt, _ = bufs[b]

            def egroup_step(eg, ecarry):
                def sub(jj, tot):
                    for k in range(4):
                        j = jj * 4 + k
                        e = eg * _L + j
                        acc_e = None
                        acc_o = None
                        for g in range(_GP):
                            sl = pl.ds(g * _L, _L)
                            sb = plsc.bitcast(rs[e, sl], jnp.bfloat16)
                            tb = plsc.bitcast(rt[e, sl], jnp.bfloat16)
                            pe, po = plsc.unpack(
                                sb * tb, format=plsc.PackFormat.INTERLEAVED)
                            if acc_e is None:
                                acc_e = pe * wev[g]
                                acc_o = po * wov[g]
                            else:
                                acc_e = acc_e + pe * wev[g]
                                acc_o = acc_o + po * wov[g]
                        s = jnp.sum(acc_e + acc_o)
                        tot = jnp.where(lane == j, s, tot)
                    return tot

                tot = lax.fori_loop(0, _L // 4, sub,
                                    jnp.zeros((_L,), jnp.float32))
                x = tot + bvec
                out_sl[c, pl.ds(eg * _L, _L)] = 1.0 / (1.0 + jnp.exp(-x))
                return ecarry

            lax.fori_loop(0, _K // _L, egroup_step, 0)

        issue(0, 0)

        def outer(i, carry):
            for b in range(2):
                c = i * 2 + b

                @pl.when(c + 1 < my_n)
                def _():
                    issue(c + 1, 1 - b)

                wait(c, b)
                compute(c, b)
            return carry

        lax.fori_loop(0, my_n // 2, outer, 0)

        @pl.when(on_fast)
        def _():
            pltpu.sync_copy(out_sl, out_r.at[pl.ds(start, _F)])

        @pl.when(jnp.logical_not(on_fast))
        def _():
            pltpu.sync_copy(out_sl.at[pl.ds(0, _S)],
                            out_r.at[pl.ds(start, _S)])

    return body(zp, zc, src2d, tgt2d, w_flat, b_vec)


def _pack_table(z):
    z16 = z.astype(jnp.bfloat16)
    return lax.bitcast_convert_type(
        z16.reshape(z.shape[0], _DW, 2), jnp.int32)


def kernel(z_protein, z_class, edge_label_index, W, b):
    n_edges = edge_label_index.shape[1]
    n_pad = _NS * _CPP * _K
    src = edge_label_index[0].astype(jnp.int32)
    tgt = edge_label_index[1].astype(jnp.int32)
    pad = jnp.zeros((n_pad - n_edges,), jnp.int32)
    src2d = jnp.concatenate([src, pad]).reshape(n_pad // _K, _K)
    tgt2d = jnp.concatenate([tgt, pad]).reshape(n_pad // _K, _K)
    zp_packed = _pack_table(z_protein)
    zc_packed = _pack_table(z_class)
    w128 = W.reshape(_D).astype(jnp.float32)
    w_flat = jnp.concatenate([w128[0::2], w128[1::2]])
    b_vec = jnp.broadcast_to(b.reshape(()), (_L,)).astype(jnp.float32)
    out2d = _sc_link_predict(zp_packed, zc_packed, src2d, tgt2d, w_flat, b_vec)
    return out2d.reshape(n_pad)[:n_edges]


# speedup vs baseline: 1.0465x; 1.0465x over previous
import functools

import jax
import jax.numpy as jnp
from jax import lax
from jax.experimental import pallas as pl
from jax.experimental.pallas import tpu as pltpu
from jax.experimental.pallas import tpu_sc as plsc

_NC = 2
_NS = 16
_NW = _NC * _NS
_L = 16
_K = 128
_D = 128
_DW = _D // 2
_GP = _DW // _L
_CPP = 160
_F = 120
_S = _CPP - _F
_FAST_CORE = 0


def _sc_link_predict(zp, zc, src2d, tgt2d, w_flat, b_vec):
    n_chunks = src2d.shape[0]

    mesh = plsc.VectorSubcoreMesh(core_axis_name="c", subcore_axis_name="s")

    @functools.partial(
        pl.kernel,
        mesh=mesh,
        out_type=jax.ShapeDtypeStruct((n_chunks, _K), jnp.float32),
        compiler_params=pltpu.CompilerParams(needs_layout_passes=False,
                                             use_tc_tiling_on_sc=False),
        scratch_types=[
            pltpu.VMEM((_F, _K), jnp.int32),
            pltpu.VMEM((_F, _K), jnp.int32),
            pltpu.VMEM((_K, _DW), jnp.int32),
            pltpu.VMEM((_K, _DW), jnp.int32),
            pltpu.VMEM((_K, _DW), jnp.int32),
            pltpu.VMEM((_K, _DW), jnp.int32),
            pltpu.VMEM((_K, _DW), jnp.int32),
            pltpu.VMEM((_K, _DW), jnp.int32),
            pltpu.VMEM((_K, _DW), jnp.int32),
            pltpu.VMEM((_K, _DW), jnp.int32),
            pltpu.VMEM((_F, _K), jnp.float32),
            pltpu.VMEM((_D,), jnp.float32),
            pltpu.VMEM((_L,), jnp.float32),
            pltpu.SemaphoreType.DMA,
            pltpu.SemaphoreType.DMA,
            pltpu.SemaphoreType.DMA,
            pltpu.SemaphoreType.DMA,
        ],
    )
    def body(zp_r, zc_r, src_r, tgt_r, w_r, b_r, out_r,
             idx_s_sl, idx_t_sl, rs0, rs1, rs2, rs3, rt0, rt1, rt2, rt3,
             out_sl, w_v, b_v, sem0, sem1, sem2, sem3):
        cid = lax.axis_index("c")
        sid = lax.axis_index("s")
        on_fast = cid == _FAST_CORE
        my_n = jnp.where(on_fast, _F, _S)
        start = pl.multiple_of(sid * _CPP + jnp.where(on_fast, 0, _F), 8)

        pltpu.sync_copy(w_r, w_v)
        pltpu.sync_copy(b_r, b_v)
        bvec = b_v[...]
        lane = lax.iota(jnp.int32, _L)
        wev = [w_v[pl.ds(g * _L, _L)] for g in range(_GP)]
        wov = [w_v[pl.ds(_DW + g * _L, _L)] for g in range(_GP)]

        @pl.when(on_fast)
        def _():
            pltpu.sync_copy(src_r.at[pl.ds(start, _F)], idx_s_sl)
            pltpu.sync_copy(tgt_r.at[pl.ds(start, _F)], idx_t_sl)

        @pl.when(jnp.logical_not(on_fast))
        def _():
            pltpu.sync_copy(src_r.at[pl.ds(start, _S)],
                            idx_s_sl.at[pl.ds(0, _S)])
            pltpu.sync_copy(tgt_r.at[pl.ds(start, _S)],
                            idx_t_sl.at[pl.ds(0, _S)])

        bufs = ((rs0, rt0, sem0), (rs1, rt1, sem1),
                (rs2, rt2, sem2), (rs3, rt3, sem3))

        def issue(c, b):
            rs, rt, sem = bufs[b]
            pltpu.async_copy(zp_r.at[idx_s_sl.at[c]], rs, sem)
            pltpu.async_copy(zc_r.at[idx_t_sl.at[c]], rt, sem)

        def wait(c, b):
            rs, rt, sem = bufs[b]
            pltpu.make_async_copy(zp_r.at[idx_s_sl.at[c]], rs, sem).wait()
            pltpu.make_async_copy(zc_r.at[idx_t_sl.at[c]], rt, sem).wait()

        def compute(c, b):
            rs, rt, _ = bufs[b]

            def egroup_step(eg, ecarry):
                def sub(jj, tot):
                    for k in range(4):
                        j = jj * 4 + k
                        e = eg * _L + j
                        acc_e = None
                        acc_o = None
                        for g in range(_GP):
                            sl = pl.ds(g * _L, _L)
                            sb = plsc.bitcast(rs[e, sl], jnp.bfloat16)
                            tb = plsc.bitcast(rt[e, sl], jnp.bfloat16)
                            pe, po = plsc.unpack(
                                sb * tb, format=plsc.PackFormat.INTERLEAVED)
                            if acc_e is None:
                                acc_e = pe * wev[g]
                                acc_o = po * wov[g]
                            else:
                                acc_e = acc_e + pe * wev[g]
                                acc_o = acc_o + po * wov[g]
                        s = jnp.sum(acc_e + acc_o)
                        tot = jnp.where(lane == j, s, tot)
                    return tot

                tot = lax.fori_loop(0, _L // 4, sub,
                                    jnp.zeros((_L,), jnp.float32))
                x = tot + bvec
                out_sl[c, pl.ds(eg * _L, _L)] = 1.0 / (1.0 + jnp.exp(-x))
                return ecarry

            lax.fori_loop(0, _K // _L, egroup_step, 0)

        issue(0, 0)
        issue(1, 1)
        issue(2, 2)

        def outer(i, carry):
            for b in range(4):
                c = i * 4 + b

                @pl.when(c + 3 < my_n)
                def _():
                    issue(c + 3, (b + 3) % 4)

                wait(c, b)
                compute(c, b)
            return carry

        lax.fori_loop(0, my_n // 4, outer, 0)

        @pl.when(on_fast)
        def _():
            pltpu.sync_copy(out_sl, out_r.at[pl.ds(start, _F)])

        @pl.when(jnp.logical_not(on_fast))
        def _():
            pltpu.sync_copy(out_sl.at[pl.ds(0, _S)],
                            out_r.at[pl.ds(start, _S)])

    return body(zp, zc, src2d, tgt2d, w_flat, b_vec)


def _pack_table(z):
    z16 = z.astype(jnp.bfloat16)
    return lax.bitcast_convert_type(
        z16.reshape(z.shape[0], _DW, 2), jnp.int32)


def kernel(z_protein, z_class, edge_label_index, W, b):
    n_edges = edge_label_index.shape[1]
    n_pad = _NS * _CPP * _K
    src = edge_label_index[0].astype(jnp.int32)
    tgt = edge_label_index[1].astype(jnp.int32)
    pad = jnp.zeros((n_pad - n_edges,), jnp.int32)
    src2d = jnp.concatenate([src, pad]).reshape(n_pad // _K, _K)
    tgt2d = jnp.concatenate([tgt, pad]).reshape(n_pad // _K, _K)
    zp_packed = _pack_table(z_protein)
    zc_packed = _pack_table(z_class)
    w128 = W.reshape(_D).astype(jnp.float32)
    w_flat = jnp.concatenate([w128[0::2], w128[1::2]])
    b_vec = jnp.broadcast_to(b.reshape(()), (_L,)).astype(jnp.float32)
    out2d = _sc_link_predict(zp_packed, zc_packed, src2d, tgt2d, w_flat, b_vec)
    return out2d.reshape(n_pad)[:n_edges]
